# baseline (speedup 1.0000x reference)
import functools
import math

import jax
import jax.numpy as jnp
from jax import lax
from jax.experimental import pallas as pl
from jax.experimental.pallas import tpu as pltpu
from jax.experimental.pallas import tpu_sc as plsc

_L = 16
_CH = 128
_NBUF = 4


def _emb_kernel(n_rows, d):
    info = plsc.get_sparse_core_info()
    nc, ns = info.num_cores, info.num_subcores
    nw = nc * ns
    b_per_w = n_rows // nw
    nchunks = b_per_w // _CH
    assert n_rows == nw * nchunks * _CH and d % _L == 0
    assert nchunks % _NBUF == 0 and nchunks // _NBUF >= 3
    scale = math.sqrt(d)

    mesh = plsc.VectorSubcoreMesh(core_axis_name="c", subcore_axis_name="s")

    @functools.partial(
        pl.kernel,
        out_type=jax.ShapeDtypeStruct((n_rows, d), jnp.float32),
        mesh=mesh,
        compiler_params=pltpu.CompilerParams(use_tc_tiling_on_sc=False),
        scratch_types=[
            pltpu.VMEM((nchunks, _CH), jnp.int32),
            [pltpu.VMEM((_CH, d), jnp.float32) for _ in range(_NBUF)],
            [pltpu.VMEM((_CH, d), jnp.float32) for _ in range(_NBUF)],
            [pltpu.SemaphoreType.DMA for _ in range(_NBUF)],
            [pltpu.SemaphoreType.DMA for _ in range(_NBUF)],
        ],
    )
    def emb(idx_hbm, table_hbm, out_hbm, idx_v, gbuf, sbuf, gsem, ssem):
        wid = lax.axis_index("s") * nc + lax.axis_index("c")
        row0 = wid * b_per_w
        pltpu.sync_copy(idx_hbm.at[pl.ds(wid * nchunks, nchunks)], idx_v)

        def start_gather(j, b):
            pltpu.async_copy(table_hbm.at[idx_v.at[j]], gbuf[b], gsem[b])

        def wait_gather(b):
            pltpu.make_async_copy(
                table_hbm.at[idx_v.at[0]], gbuf[b], gsem[b]).wait()

        def start_scatter(j, b):
            pltpu.async_copy(
                sbuf[b], out_hbm.at[pl.ds(row0 + j * _CH, _CH)], ssem[b])

        def wait_scatter(b):
            pltpu.make_async_copy(
                sbuf[b], out_hbm.at[pl.ds(row0, _CH)], ssem[b]).wait()

        def do_scale(b):
            src, dst = gbuf[b], sbuf[b]

            @pl.loop(0, _CH)
            def _(r):
                for c in range(d // _L):
                    sl = pl.ds(c * _L, _L)
                    dst[r, sl] = src[r, sl] * scale

        for b in range(_NBUF):
            start_gather(b, b)
        for b in range(_NBUF):
            wait_gather(b)
            do_scale(b)
            start_scatter(b, b)
            start_gather(b + _NBUF, b)
        @pl.loop(_NBUF, nchunks - _NBUF, step=_NBUF)
        def _(j0):
            for b in range(_NBUF):
                j = j0 + b
                wait_scatter(b)
                wait_gather(b)
                do_scale(b)
                start_scatter(j, b)
                start_gather(j + _NBUF, b)
        for b in range(_NBUF):
            j = nchunks - _NBUF + b
            wait_scatter(b)
            wait_gather(b)
            do_scale(b)
            start_scatter(j, b)
        for b in range(_NBUF):
            wait_scatter(b)

    return emb


def kernel(x, table):
    n_rows = x.size
    d = table.shape[1]
    idx = x.reshape(n_rows // _CH, _CH).astype(jnp.int32)
    out = _emb_kernel(n_rows, d)(idx, table)
    return out.reshape(x.shape + (d,))
